# baseline (speedup 1.0000x reference)
import functools

import jax
import jax.numpy as jnp
from jax import lax
from jax.experimental import pallas as pl
from jax.experimental.pallas import tpu as pltpu
from jax.experimental.pallas import tpu_sc as plsc

DIM = 64
SLEN = 50
NULL_IDX = 0
MAX_NORM = 10.0
_NC = 2
_NS = 16
_NW = _NC * _NS
_NB = 8
_L = 16

_GATHER_DNUMS = lax.GatherDimensionNumbers(
    offset_dims=(), collapsed_slice_dims=(0,), start_index_map=(0,))


def _shuffle(x, perm):
    return lax.gather(x, perm[:, None], _GATHER_DNUMS, (1,),
                      mode=lax.GatherScatterMode.PROMISE_IN_BOUNDS)


def _lane_sum(x):
    lane = lax.iota(jnp.int32, _L)
    for sh in (1, 2, 4, 8):
        x = x + _shuffle(x, lane ^ sh)
    return x


def _pooled_embed(table, idx):
    bags = idx.shape[0]
    bpw = bags // _NW
    nchunk = bpw // _NB
    npair = nchunk // 2
    mesh = plsc.VectorSubcoreMesh(core_axis_name="c", subcore_axis_name="s")

    @functools.partial(
        pl.kernel,
        mesh=mesh,
        out_type=jax.ShapeDtypeStruct((bags, DIM), jnp.float32),
        scratch_types=[
            pltpu.VMEM((bpw, SLEN), jnp.int32),
            pltpu.VMEM((2, _NB, SLEN, DIM), jnp.float32),
            pltpu.VMEM((bpw, DIM), jnp.float32),
            pltpu.SemaphoreType.DMA,
            pltpu.SemaphoreType.DMA,
        ],
        compiler_params=pltpu.CompilerParams(use_tc_tiling_on_sc=False),
    )
    def k(table_hbm, idx_hbm, out_hbm, idx_v, rows_v, out_v, sem_a, sem_b):
        wid = lax.axis_index("s") * _NC + lax.axis_index("c")
        base = wid * bpw
        pltpu.sync_copy(idx_hbm.at[pl.ds(base, bpw)], idx_v)

        def fire(c, b, sem):
            for j in range(_NB):
                pltpu.async_copy(table_hbm.at[idx_v.at[c * _NB + j]],
                                 rows_v.at[b, j], sem)

        def drain(b, sem):
            for j in range(_NB):
                pltpu.make_async_copy(table_hbm.at[idx_v.at[j]],
                                      rows_v.at[b, j], sem).wait()

        def compute(c, b):
            rows_b = rows_v.at[b]

            def bag_body(j, carry2):
                jj = c * _NB + j
                i0 = idx_v[jj, pl.ds(0, _L)]
                i1 = idx_v[jj, pl.ds(_L, _L)]
                i2 = idx_v[jj, pl.ds(2 * _L, _L)]
                i3 = idx_v[jj, pl.ds(SLEN - _L, _L)]
                lane = lax.iota(jnp.int32, _L)
                one = jnp.ones((_L,), jnp.float32)
                zero = jnp.zeros((_L,), jnp.float32)
                c_vec = (jnp.where(i0 != NULL_IDX, one, zero)
                         + jnp.where(i1 != NULL_IDX, one, zero)
                         + jnp.where(i2 != NULL_IDX, one, zero)
                         + jnp.where(
                             (i3 != NULL_IDX) & (lane >= 4 * _L - SLEN),
                             one, zero))
                cnt_f = _lane_sum(c_vec)

                z = jnp.zeros((_L,), jnp.float32)
                perm_a = (lane & 3) * 4

                def group_body(g, st, nrows=4):
                    acc = list(st)
                    r0 = g * 4
                    q = [[rows_b[j, r0 + k, pl.ds(v * _L, _L)]
                          for v in range(4)] if k < nrows else [z] * 4
                         for k in range(4)]
                    p = []
                    for k in range(4):
                        pk = (q[k][0] * q[k][0] + q[k][1] * q[k][1]
                              + q[k][2] * q[k][2] + q[k][3] * q[k][3])
                        pk = pk + _shuffle(pk, lane ^ 1)
                        pk = pk + _shuffle(pk, lane ^ 2)
                        p.append(_shuffle(pk, perm_a))
                    w01 = jnp.where(lane < 4, p[0], p[1])
                    w23 = jnp.where(lane < 12, p[2], p[3])
                    s = jnp.where(lane < 8, w01, w23)
                    s = s + _shuffle(s, lane ^ 1)
                    s = s + _shuffle(s, lane ^ 2)
                    s = jnp.maximum(s, 1.0)
                    bi = lax.bitcast_convert_type(s, jnp.int32)
                    bi = jnp.int32(0x5F3759DF) - lax.shift_right_arithmetic(bi, 1)
                    y = lax.bitcast_convert_type(bi, jnp.float32)
                    h = -0.5 * s
                    y = y * (1.5 + h * y * y)
                    y = y * (1.5 + h * y * y)
                    scale = jnp.minimum(1.0, MAX_NORM * y)
                    for k in range(nrows):
                        bs = _shuffle(scale, jnp.full((_L,), 4 * k, jnp.int32))
                        for v in range(4):
                            acc[v] = acc[v] + q[k][v] * bs
                    return tuple(acc)

                st = lax.fori_loop(0, SLEN // 4, group_body, (z, z, z, z),
                                   unroll=3)
                a0, a1, a2, a3 = group_body(SLEN // 4, st, nrows=SLEN % 4)
                inv = 1.0 / jnp.maximum(cnt_f, jnp.float32(1e-20))
                out_v[jj, pl.ds(0, _L)] = a0 * inv
                out_v[jj, pl.ds(_L, _L)] = a1 * inv
                out_v[jj, pl.ds(2 * _L, _L)] = a2 * inv
                out_v[jj, pl.ds(3 * _L, _L)] = a3 * inv
                return carry2

            lax.fori_loop(0, _NB, bag_body, 0)

        fire(0, 0, sem_a)

        def pair_body(i, carry):
            c0 = 2 * i
            fire(c0 + 1, 1, sem_b)
            drain(0, sem_a)
            compute(c0, 0)

            @pl.when(i < npair - 1)
            def _():
                fire(c0 + 2, 0, sem_a)

            drain(1, sem_b)
            compute(c0 + 1, 1)
            return carry

        lax.fori_loop(0, npair, pair_body, 0)
        pltpu.sync_copy(out_v, out_hbm.at[pl.ds(base, bpw)])

    return k(table, idx)


def kernel(xs, ys, table):
    idx = jnp.concatenate([xs, ys], axis=0)
    out = _pooled_embed(table, idx)
    b = xs.shape[0]
    return out[:b, None, :], out[b:, None, :]


# speedup vs baseline: 1.0919x; 1.0919x over previous
import functools

import jax
import jax.numpy as jnp
from jax import lax
from jax.experimental import pallas as pl
from jax.experimental.pallas import tpu as pltpu
from jax.experimental.pallas import tpu_sc as plsc

DIM = 64
SLEN = 50
NULL_IDX = 0
MAX_NORM = 10.0
_NC = 2
_NS = 16
_NW = _NC * _NS
_NB = 8
_L = 16

_GATHER_DNUMS = lax.GatherDimensionNumbers(
    offset_dims=(), collapsed_slice_dims=(0,), start_index_map=(0,))


def _shuffle(x, perm):
    return lax.gather(x, perm[:, None], _GATHER_DNUMS, (1,),
                      mode=lax.GatherScatterMode.PROMISE_IN_BOUNDS)


def _lane_sum(x):
    lane = lax.iota(jnp.int32, _L)
    for sh in (1, 2, 4, 8):
        x = x + _shuffle(x, lane ^ sh)
    return x


def _pooled_embed(table, idx):
    bags = idx.shape[0]
    bpw = bags // _NW
    nchunk = bpw // _NB
    npair = nchunk // 2
    mesh = plsc.VectorSubcoreMesh(core_axis_name="c", subcore_axis_name="s")

    @functools.partial(
        pl.kernel,
        mesh=mesh,
        out_type=jax.ShapeDtypeStruct((bags, DIM), jnp.float32),
        scratch_types=[
            pltpu.VMEM((bpw, SLEN), jnp.int32),
            pltpu.VMEM((2, _NB, SLEN, DIM), jnp.float32),
            pltpu.VMEM((bpw, DIM), jnp.float32),
            pltpu.SemaphoreType.DMA,
            pltpu.SemaphoreType.DMA,
        ],
        compiler_params=pltpu.CompilerParams(use_tc_tiling_on_sc=False),
    )
    def k(table_hbm, idx_hbm, out_hbm, idx_v, rows_v, out_v, sem_a, sem_b):
        wid = lax.axis_index("s") * _NC + lax.axis_index("c")
        base = wid * bpw
        pltpu.sync_copy(idx_hbm.at[pl.ds(base, bpw)], idx_v)

        def fire(c, b, sem):
            for j in range(_NB):
                pltpu.async_copy(table_hbm.at[idx_v.at[c * _NB + j]],
                                 rows_v.at[b, j], sem)

        def drain(b, sem):
            for j in range(_NB):
                pltpu.make_async_copy(table_hbm.at[idx_v.at[j]],
                                      rows_v.at[b, j], sem).wait()

        def compute(c, b):
            rows_b = rows_v.at[b]

            def bag_body(j, carry2):
                jj = c * _NB + j
                i0 = idx_v[jj, pl.ds(0, _L)]
                i1 = idx_v[jj, pl.ds(_L, _L)]
                i2 = idx_v[jj, pl.ds(2 * _L, _L)]
                i3 = idx_v[jj, pl.ds(SLEN - _L, _L)]
                lane = lax.iota(jnp.int32, _L)
                one = jnp.ones((_L,), jnp.float32)
                zero = jnp.zeros((_L,), jnp.float32)
                c_vec = (jnp.where(i0 != NULL_IDX, one, zero)
                         + jnp.where(i1 != NULL_IDX, one, zero)
                         + jnp.where(i2 != NULL_IDX, one, zero)
                         + jnp.where(
                             (i3 != NULL_IDX) & (lane >= 4 * _L - SLEN),
                             one, zero))
                cnt_f = _lane_sum(c_vec)

                def row_body(r, st):
                    a0, a1, a2, a3 = st
                    r0 = rows_b[j, r, pl.ds(0, _L)]
                    r1 = rows_b[j, r, pl.ds(_L, _L)]
                    r2 = rows_b[j, r, pl.ds(2 * _L, _L)]
                    r3 = rows_b[j, r, pl.ds(3 * _L, _L)]
                    s = _lane_sum(r0 * r0 + r1 * r1 + r2 * r2 + r3 * r3)
                    s = jnp.maximum(s, 1.0)
                    bi = lax.bitcast_convert_type(s, jnp.int32)
                    bi = jnp.int32(0x5F375A86) - lax.shift_right_arithmetic(bi, 1)
                    y = lax.bitcast_convert_type(bi, jnp.float32)
                    h = -0.5 * s
                    y = y * (1.5 + h * y * y)
                    scale = jnp.minimum(1.0, MAX_NORM * y)
                    return (a0 + r0 * scale, a1 + r1 * scale,
                            a2 + r2 * scale, a3 + r3 * scale)

                z = jnp.zeros((_L,), jnp.float32)
                a0, a1, a2, a3 = lax.fori_loop(
                    0, SLEN, row_body, (z, z, z, z), unroll=5)
                inv = 1.0 / jnp.maximum(cnt_f, jnp.float32(1e-20))
                out_v[jj, pl.ds(0, _L)] = a0 * inv
                out_v[jj, pl.ds(_L, _L)] = a1 * inv
                out_v[jj, pl.ds(2 * _L, _L)] = a2 * inv
                out_v[jj, pl.ds(3 * _L, _L)] = a3 * inv
                return carry2

            lax.fori_loop(0, _NB, bag_body, 0)

        fire(0, 0, sem_a)

        def pair_body(i, carry):
            c0 = 2 * i
            fire(c0 + 1, 1, sem_b)
            drain(0, sem_a)
            compute(c0, 0)

            @pl.when(i < npair - 1)
            def _():
                fire(c0 + 2, 0, sem_a)

            drain(1, sem_b)
            compute(c0 + 1, 1)
            return carry

        lax.fori_loop(0, npair, pair_body, 0)
        pltpu.sync_copy(out_v, out_hbm.at[pl.ds(base, bpw)])

    return k(table, idx)


def kernel(xs, ys, table):
    idx = jnp.concatenate([xs, ys], axis=0)
    out = _pooled_embed(table, idx)
    b = xs.shape[0]
    return out[:b, None, :], out[b:, None, :]
